# baseline (speedup 1.0000x reference)
import functools

import jax
import jax.numpy as jnp
from jax import lax
from jax.experimental import pallas as pl
from jax.experimental.pallas import tpu as pltpu
from jax.experimental.pallas import tpu_sc as plsc

N = 10000
E = 160000
NC = 2
NS = 16
NW = NC * NS
NR = 320
NPAD = NW * NR
FLUSH = 8192
PKCAP = 168192
CG = 128
NFULL = 39
NEXTRA = 2
EID_MASK = 0x3FFFF
NRL = N - (NW - 1) * NR
NEGINF = float("-inf")

_mesh = plsc.VectorSubcoreMesh(core_axis_name="c", subcore_axis_name="s",
                               num_cores=NC, num_subcores=NS)
_sc_params = pltpu.CompilerParams(use_tc_tiling_on_sc=False,
                                  needs_layout_passes=False)
_sc_params_tc = pltpu.CompilerParams(use_tc_tiling_on_sc=True,
                                     needs_layout_passes=False)


def _wid():
    return lax.axis_index("s") * NC + lax.axis_index("c")


def _f32(shape):
    return jax.ShapeDtypeStruct(shape, jnp.float32)


def _i32(shape):
    return jax.ShapeDtypeStruct(shape, jnp.int32)


@functools.partial(
    pl.kernel,
    out_type=(_i32((NW, PKCAP)), _i32((NW, 16))),
    mesh=_mesh,
    compiler_params=_sc_params,
    scratch_types=[
        pltpu.VMEM((8000,), jnp.int32),
        pltpu.VMEM((FLUSH + 16,), jnp.int32),
        pltpu.VMEM((16,), jnp.int32),
    ],
)
def _route(dst_hbm, pk_hbm, cnt_hbm, dstbuf, pkbuf, cntbuf):
    wid = _wid()
    lo = wid * NR
    hi = lo + NR
    CH = 8000

    def chunk_body(c, carry):
        ptr0, off0 = carry
        pltpu.sync_copy(dst_hbm.at[pl.ds(pl.multiple_of(c * CH, CH), CH)],
                        dstbuf)

        def vec_body(j, carry2):
            ptr, off = carry2
            v = dstbuf[pl.ds(j * 16, 16)]
            eidv = (c * CH + j * 16) + lax.iota(jnp.int32, 16)
            mask = (v >= lo) & (v < hi)
            packed = eidv | ((v - lo) << 18)
            mv = mask.astype(jnp.int32)
            cum = plsc.cumsum(mv)
            plsc.store_scatter(pkbuf, [ptr + cum - mv], packed, mask=mask)
            ptr = ptr + cum[15]
            flush = ptr >= FLUSH

            @pl.when(flush)
            def _():
                pltpu.sync_copy(
                    pkbuf.at[pl.ds(0, FLUSH)],
                    pk_hbm.at[wid, pl.ds(pl.multiple_of(off, FLUSH), FLUSH)])
                tail = pkbuf[pl.ds(FLUSH, 16)]
                pkbuf[pl.ds(0, 16)] = tail

            ptr = jnp.where(flush, ptr - FLUSH, ptr)
            off = jnp.where(flush, off + FLUSH, off)
            return ptr, off

        return lax.fori_loop(0, CH // 16, vec_body, (ptr0, off0))

    ptr, off = lax.fori_loop(0, E // CH, chunk_body,
                             (jnp.int32(0), jnp.int32(0)))
    pltpu.sync_copy(pkbuf.at[pl.ds(0, FLUSH)],
                    pk_hbm.at[wid, pl.ds(pl.multiple_of(off, FLUSH), FLUSH)])
    cntbuf[...] = jnp.full((16,), off + ptr, jnp.int32)
    pltpu.sync_copy(cntbuf, cnt_hbm.at[wid])


def _make_gather2(d0, d1, tc_tiled=False):
    @functools.partial(
        pl.kernel,
        out_type=(_f32((E, d0)), _f32((E, d1))),
        mesh=_mesh,
        compiler_params=_sc_params_tc if tc_tiled else _sc_params,
        scratch_types=[
            pltpu.VMEM((CG,), jnp.int32),
            pltpu.VMEM((CG,), jnp.int32),
            pltpu.VMEM((CG, d0), jnp.float32),
            pltpu.VMEM((CG, d1), jnp.float32),
            pltpu.SemaphoreType.DMA,
            pltpu.SemaphoreType.DMA,
        ],
    )
    def gather2(t0, i0, t1, i1, o0, o1, idxv0, idxv1, rows0, rows1,
                sem0, sem1):
        wid = _wid()

        def do(b):
            base = pl.multiple_of(b, CG)
            pltpu.sync_copy(i0.at[pl.ds(base, CG)], idxv0)
            pltpu.sync_copy(i1.at[pl.ds(base, CG)], idxv1)
            d0 = pltpu.async_copy(t0.at[idxv0], rows0, sem0)
            d1 = pltpu.async_copy(t1.at[idxv1], rows1, sem1)
            d0.wait()
            pltpu.sync_copy(rows0, o0.at[pl.ds(base, CG)])
            d1.wait()
            pltpu.sync_copy(rows1, o1.at[pl.ds(base, CG)])

        def body(c, _):
            do((wid * NFULL + c) * CG)
            return 0

        lax.fori_loop(0, NFULL, body, 0)

        @pl.when(wid < NEXTRA)
        def _():
            do((NW * NFULL + wid) * CG)

    return gather2


def _make_gather1(d0):
    @functools.partial(
        pl.kernel,
        out_type=_f32((E, d0)),
        mesh=_mesh,
        compiler_params=_sc_params,
        scratch_types=[
            pltpu.VMEM((CG,), jnp.int32),
            pltpu.VMEM((CG, d0), jnp.float32),
            pltpu.SemaphoreType.DMA,
        ],
    )
    def gather1(t0, i0, o0, idxv, rows0, sem):
        wid = _wid()

        def do(b):
            base = pl.multiple_of(b, CG)
            pltpu.sync_copy(i0.at[pl.ds(base, CG)], idxv)
            pltpu.async_copy(t0.at[idxv], rows0, sem).wait()
            pltpu.sync_copy(rows0, o0.at[pl.ds(base, CG)])

        def body(c, _):
            do((wid * NFULL + c) * CG)
            return 0

        lax.fori_loop(0, NFULL, body, 0)

        @pl.when(wid < NEXTRA)
        def _():
            do((NW * NFULL + wid) * CG)

    return gather1


_gather_16_16 = _make_gather2(16, 16)
_gather_64_64 = _make_gather2(64, 64)
_gather_128_128 = _make_gather2(128, 128, tc_tiled=True)
_gather_32 = _make_gather1(32)


def _load_chunk(pk_hbm, wid, c, pkchunk, eidbuf):
    pltpu.sync_copy(pk_hbm.at[wid, pl.ds(pl.multiple_of(c * CG, CG), CG)],
                    pkchunk.at[pl.ds(0, CG)])
    for j in range(CG // 16):
        pv = pkchunk[pl.ds(j * 16, 16)]
        ev = jnp.minimum(pv & EID_MASK, jnp.int32(E - 1))
        eidbuf[pl.ds(j * 16, 16)] = ev


def _make_scatmax(d, tc_tiled=False):
    ngr = d // 16

    @functools.partial(
        pl.kernel,
        out_type=_f32((N, d)),
        mesh=_mesh,
        compiler_params=_sc_params_tc if tc_tiled else _sc_params,
        scratch_types=[
            pltpu.VMEM((NR, d), jnp.float32),
            pltpu.VMEM((CG + 16,), jnp.int32),
            pltpu.VMEM((CG,), jnp.int32),
            pltpu.VMEM((CG, d), jnp.float32),
            pltpu.VMEM((16,), jnp.int32),
            pltpu.SemaphoreType.DMA,
        ],
    )
    def scatmax(m_hbm, pk_hbm, cnt_hbm, out_hbm, acc, pkchunk, eidbuf,
                rows, cntv, sem):
        wid = _wid()

        def init_body(r, _):
            for g in range(ngr):
                acc[r, pl.ds(g * 16, 16)] = jnp.full((16,), NEGINF)
            return 0

        lax.fori_loop(0, NR, init_body, 0)

        pltpu.sync_copy(cnt_hbm.at[wid], cntv)
        cnt = cntv[...][0]
        nch = (cnt + CG - 1) // CG

        def chunk_body(c, _):
            _load_chunk(pk_hbm, wid, c, pkchunk, eidbuf)
            pltpu.async_copy(m_hbm.at[eidbuf], rows, sem).wait()
            jn = jnp.minimum(jnp.int32(CG), cnt - c * CG)

            def edge_body(j, _2):
                dl = pkchunk[pl.ds(j, 16)][0] >> 18
                for g in range(ngr):
                    sl = pl.ds(g * 16, 16)
                    acc[dl, sl] = jnp.maximum(acc[dl, sl], rows[j, sl])
                return 0

            lax.fori_loop(0, jn, edge_body, 0)
            return 0

        lax.fori_loop(0, nch, chunk_body, 0)

        def fix_body(r, _):
            for g in range(ngr):
                sl = pl.ds(g * 16, 16)
                v = acc[r, sl]
                acc[r, sl] = jnp.where(v == NEGINF, jnp.float32(0.0), v)
            return 0

        lax.fori_loop(0, NR, fix_body, 0)
        last = wid == NW - 1

        @pl.when(last)
        def _():
            pltpu.sync_copy(acc.at[pl.ds(0, NRL)],
                            out_hbm.at[pl.ds(wid * NR, NRL)])

        @pl.when(jnp.logical_not(last))
        def _():
            pltpu.sync_copy(acc, out_hbm.at[pl.ds(wid * NR, NR)])

    return scatmax


_scatmax64 = _make_scatmax(64)
_scatmax128 = _make_scatmax(128, tc_tiled=True)
_scatmax192 = _make_scatmax(192)


@functools.partial(
    pl.kernel,
    out_type=_f32((N, 32)),
    mesh=_mesh,
    compiler_params=_sc_params,
    scratch_types=[
        pltpu.VMEM((NR, 32), jnp.float32),
        pltpu.VMEM((CG + 16,), jnp.int32),
        pltpu.VMEM((CG,), jnp.int32),
        pltpu.VMEM((CG, 16), jnp.float32),
        pltpu.VMEM((16,), jnp.int32),
        pltpu.SemaphoreType.DMA,
    ],
)
def _stats(a_hbm, pk_hbm, cnt_hbm, out_hbm, acc, pkchunk, eidbuf, rows,
           cntv, sem):
    wid = _wid()
    lo_m = pl.ds(0, 16)
    lo_s = pl.ds(16, 16)

    def init_body(r, _):
        acc[r, lo_m] = jnp.full((16,), NEGINF)
        acc[r, lo_s] = jnp.zeros((16,), jnp.float32)
        return 0

    lax.fori_loop(0, NR, init_body, 0)

    pltpu.sync_copy(cnt_hbm.at[wid], cntv)
    cnt = cntv[...][0]
    nch = (cnt + CG - 1) // CG

    def max_chunk(c, _):
        _load_chunk(pk_hbm, wid, c, pkchunk, eidbuf)
        pltpu.async_copy(a_hbm.at[eidbuf], rows, sem).wait()
        jn = jnp.minimum(jnp.int32(CG), cnt - c * CG)

        def edge_body(j, _2):
            dl = pkchunk[pl.ds(j, 16)][0] >> 18
            acc[dl, lo_m] = jnp.maximum(acc[dl, lo_m], rows[j, pl.ds(0, 16)])
            return 0

        lax.fori_loop(0, jn, edge_body, 0)
        return 0

    lax.fori_loop(0, nch, max_chunk, 0)

    def sum_chunk(c, _):
        _load_chunk(pk_hbm, wid, c, pkchunk, eidbuf)
        pltpu.async_copy(a_hbm.at[eidbuf], rows, sem).wait()
        jn = jnp.minimum(jnp.int32(CG), cnt - c * CG)

        def edge_body(j, _2):
            dl = pkchunk[pl.ds(j, 16)][0] >> 18
            e = jnp.exp(rows[j, pl.ds(0, 16)] - acc[dl, lo_m])
            acc[dl, lo_s] = acc[dl, lo_s] + e
            return 0

        lax.fori_loop(0, jn, edge_body, 0)
        return 0

    lax.fori_loop(0, nch, sum_chunk, 0)
    last = wid == NW - 1

    @pl.when(last)
    def _():
        pltpu.sync_copy(acc.at[pl.ds(0, NRL)],
                        out_hbm.at[pl.ds(wid * NR, NRL)])

    @pl.when(jnp.logical_not(last))
    def _():
        pltpu.sync_copy(acc, out_hbm.at[pl.ds(wid * NR, NR)])


EBLK = 1000
NEB = E // EBLK


def _eb(d):
    return pl.BlockSpec((EBLK, d), lambda i: (i, 0))


def _full2(shape):
    return pl.BlockSpec(shape, lambda i: (0, 0))


def _dot(a, b):
    return jnp.dot(a, b, preferred_element_type=jnp.float32)


def _bn_relu(v, w, b):
    mu = jnp.mean(v, axis=0, keepdims=True)
    var = jnp.mean((v - mu) * (v - mu), axis=0, keepdims=True)
    return jax.nn.relu((v - mu) / jnp.sqrt(var + 1e-5) * w + b)


def _node_prep_body(x_ref, wn_ref, bn_ref, ne_ref):
    ne_ref[...] = jax.nn.relu(
        _dot(x_ref[...] * 0.01, wn_ref[...]) + bn_ref[...])


def _node_prep(x, wn, bnb):
    return pl.pallas_call(
        _node_prep_body,
        out_shape=_f32((N, 16)),
    )(x, wn, bnb)


def _make_mix(d_in, d_out):
    def body(gd_ref, gs_ref, w1_ref, b1_ref, w2_ref, b2_ref, m_ref):
        xi = gd_ref[...]
        xj = gs_ref[...]
        m_in = jnp.concatenate([xi, xj - xi], axis=1)
        m = jax.nn.relu(_dot(m_in, w1_ref[...]) + b1_ref[...])
        m_ref[...] = _dot(m, w2_ref[...]) + b2_ref[...]

    def mix(gd, gs, w1, b1, w2, b2):
        return pl.pallas_call(
            body,
            grid=(NEB,),
            in_specs=[_eb(d_in), _eb(d_in),
                      _full2((2 * d_in, d_out)), _full2((1, d_out)),
                      _full2((d_out, d_out)), _full2((1, d_out))],
            out_specs=_eb(d_out),
            out_shape=_f32((E, d_out)),
        )(gd, gs, w1, b1, w2, b2)

    return mix


_mix1 = _make_mix(16, 64)
_mix2 = _make_mix(64, 128)


def _node_mid_body(c1_ref, bw_ref, bb_ref, c1o_ref):
    c1o_ref[...] = _bn_relu(c1_ref[...], bw_ref[...], bb_ref[...])


def _node_mid(c1raw, bw, bb):
    return pl.pallas_call(
        _node_mid_body,
        out_shape=_f32((N, 64)),
    )(c1raw, bw, bb)


def _node_mid2_body(c2_ref, bw_ref, bb_ref, lx3_ref, lxb3_ref, lx4_ref,
                    lxb4_ref, hxt_ref):
    h = _bn_relu(c2_ref[...], bw_ref[...], bb_ref[...])
    hx3 = _dot(h, lx3_ref[...]) + lxb3_ref[...]
    hx4 = _dot(h, lx4_ref[...]) + lxb4_ref[...]
    hxt_ref[...] = jnp.concatenate([hx3, hx4], axis=1)


def _node_mid2(c2raw, bw, bb, lx3, lxb3, lx4, lxb4):
    return pl.pallas_call(
        _node_mid2_body,
        out_shape=_f32((N, 128)),
    )(c2raw, bw, bb, lx3, lxb3, lx4, lxb4)


def _edge_e2(ea_ref, we_ref, be_ref, em_ref):
    ea = ea_ref[...]
    ea = jnp.concatenate([ea[:, :1] * 0.01, ea[:, 1:]], axis=1)
    ee = jax.nn.relu(_dot(ea, we_ref[...]) + be_ref[...])
    return jax.nn.leaky_relu(_dot(ee, em_ref[...]), 0.2)


def _attn_body(ea_ref, ghd_ref, ghs_ref, we_ref, be_ref, em3_ref, em4_ref,
               at3_ref, at4_ref, a_ref):
    ghd = ghd_ref[...]
    ghs = ghs_ref[...]
    e23 = _edge_e2(ea_ref, we_ref, be_ref, em3_ref)
    e24 = _edge_e2(ea_ref, we_ref, be_ref, em4_ref)
    c3 = jnp.concatenate([ghd[:, :64], ghs[:, :64], e23], axis=1)
    c4 = jnp.concatenate([ghd[:, 64:], ghs[:, 64:], e24], axis=1)
    a3 = jax.nn.leaky_relu(_dot(c3, at3_ref[...]), 0.2)
    a4 = jax.nn.leaky_relu(_dot(c4, at4_ref[...]), 0.2)
    a_ref[...] = jnp.concatenate(
        [a3, a4, jnp.zeros((EBLK, 10), jnp.float32)], axis=1)


def _attn(ea3, ghd, ghs, we, be, em3, em4, at3, at4):
    return pl.pallas_call(
        _attn_body,
        grid=(NEB,),
        in_specs=[_eb(3), _eb(128), _eb(128),
                  _full2((3, 64)), _full2((1, 64)),
                  _full2((64, 128)), _full2((64, 128)),
                  _full2((256, 3)), _full2((256, 3))],
        out_specs=_eb(16),
        out_shape=_f32((E, 16)),
    )(ea3, ghd, ghs, we, be, em3, em4, at3, at4)


def _wout_body(ea_ref, ghs_ref, a_ref, gst_ref, we_ref, be_ref, em3_ref,
               em4_ref, ln3_ref, ln4_ref, lb3_ref, lb4_ref,
               o3_ref, o4_ref):
    ghs = ghs_ref[...]
    e23 = _edge_e2(ea_ref, we_ref, be_ref, em3_ref)
    e24 = _edge_e2(ea_ref, we_ref, be_ref, em4_ref)
    o3 = _dot(jnp.concatenate([ghs[:, :64], e23], axis=1),
              ln3_ref[...]) + lb3_ref[...]
    o4 = _dot(jnp.concatenate([ghs[:, 64:], e24], axis=1),
              ln4_ref[...]) + lb4_ref[...]
    gst = gst_ref[...]
    w6 = jnp.exp(a_ref[...][:, :6] - gst[:, :6]) / (gst[:, 16:22] + 1e-16)
    o3_ref[...] = jnp.concatenate(
        [o3 * w6[:, 0:1], o3 * w6[:, 1:2], o3 * w6[:, 2:3]], axis=1)
    o4_ref[...] = jnp.concatenate(
        [o4 * w6[:, 3:4], o4 * w6[:, 4:5], o4 * w6[:, 5:6]], axis=1)


def _wout(ea3, ghs, a, gst, we, be, em3, em4, ln3, ln4, lb3, lb4):
    return pl.pallas_call(
        _wout_body,
        grid=(NEB,),
        in_specs=[_eb(3), _eb(128), _eb(16), _eb(32),
                  _full2((3, 64)), _full2((1, 64)),
                  _full2((64, 128)), _full2((64, 128)),
                  _full2((192, 64)), _full2((192, 64)),
                  _full2((1, 64)), _full2((1, 64))],
        out_specs=(_eb(192), _eb(192)),
        out_shape=(_f32((E, 192)), _f32((E, 192))),
    )(ea3, ghs, a, gst, we, be, em3, em4, ln3, ln4, lb3, lb4)


NBLK = 10
NBR = N // NBLK


def _nb(d):
    return pl.BlockSpec((NBR, d), lambda i: (i, 0))


def _bnstats_body(agg3_ref, agg4_ref, hxt_ref, st_ref):
    i = pl.program_id(0)

    @pl.when(i == 0)
    def _():
        st_ref[...] = jnp.zeros_like(st_ref)

    hxt = hxt_ref[...]
    hx3 = hxt[:, :64]
    hx4 = hxt[:, 64:]
    p3 = agg3_ref[...] + jnp.concatenate([hx3, hx3, hx3], axis=1)
    p4 = agg4_ref[...] + jnp.concatenate([hx4, hx4, hx4], axis=1)
    st = jnp.stack([
        jnp.sum(p3, axis=0), jnp.sum(p3 * p3, axis=0),
        jnp.sum(p4, axis=0), jnp.sum(p4 * p4, axis=0)], axis=0)
    st_ref[...] = st_ref[...] + st


def _bnstats(agg3, agg4, hxt):
    return pl.pallas_call(
        _bnstats_body,
        grid=(NBLK,),
        in_specs=[_nb(192), _nb(192), _nb(128)],
        out_specs=pl.BlockSpec((4, 192), lambda i: (0, 0)),
        out_shape=_f32((4, 192)),
    )(agg3, agg4, hxt)


def _final_body(agg3_ref, agg4_ref, hxt_ref, st_ref, b2w_ref,
                b2b_ref, b22w_ref, b22b_ref, ew_ref, eb_ref, dw_ref, db_ref,
                cls_ref, reg_ref, cco_ref, cro_ref):
    st = st_ref[...]
    mu3 = st[0:1] * (1.0 / N)
    var3 = st[1:2] * (1.0 / N) - mu3 * mu3
    mu4 = st[2:3] * (1.0 / N)
    var4 = st[3:4] * (1.0 / N) - mu4 * mu4
    hxt = hxt_ref[...]
    hx3 = hxt[:, :64]
    hx4 = hxt[:, 64:]
    p3 = agg3_ref[...] + jnp.concatenate([hx3, hx3, hx3], axis=1)
    p4 = agg4_ref[...] + jnp.concatenate([hx4, hx4, hx4], axis=1)
    cco = jax.nn.relu(
        (p3 - mu3) / jnp.sqrt(var3 + 1e-5) * b2w_ref[...] + b2b_ref[...])
    cro = jax.nn.relu(
        (p4 - mu4) / jnp.sqrt(var4 + 1e-5) * b22w_ref[...] + b22b_ref[...])
    cco_ref[...] = cco
    cro_ref[...] = cro
    emb3 = jax.nn.relu(_dot(cco, ew_ref[...]) + eb_ref[...])
    emb4 = jax.nn.relu(_dot(cro, ew_ref[...]) + eb_ref[...])
    cls_ref[...] = jax.nn.sigmoid(_dot(emb3, dw_ref[...]) + db_ref[...])
    reg_ref[...] = jax.nn.sigmoid(_dot(emb4, dw_ref[...]) + db_ref[...])


def _final(agg3, agg4, hxt, b2w, b2b, b22w, b22b, ew, ebias, dw, db):
    st = _bnstats(agg3, agg4, hxt)
    return pl.pallas_call(
        _final_body,
        grid=(NBLK,),
        in_specs=[_nb(192), _nb(192), _nb(128),
                  pl.BlockSpec((4, 192), lambda i: (0, 0)),
                  _full2((1, 192)), _full2((1, 192)),
                  _full2((1, 192)), _full2((1, 192)),
                  _full2((192, 64)), _full2((1, 64)),
                  _full2((64, 4)), _full2((1, 4))],
        out_specs=(_nb(4), _nb(4), _nb(192), _nb(192)),
        out_shape=(_f32((N, 4)), _f32((N, 4)),
                   _f32((N, 192)), _f32((N, 192))),
    )(agg3, agg4, hxt, st, b2w, b2b, b22w, b22b, ew, ebias, dw, db)


def kernel(x, edge_index, edge_attr, params):
    p = params
    src = edge_index[0]
    dst = edge_index[1]
    ea3 = edge_attr[:, 5:8]

    def r1(v):
        return v.reshape(1, -1)

    pk, cnt = _route(dst)

    ne = _node_prep(x, p['mlp_node_w'], r1(p['mlp_node_b']))
    gd1, gs1 = _gather_16_16(ne, dst, ne, src)
    m1 = _mix1(gd1, gs1, p['c1_w1'], r1(p['c1_b1']),
               p['c1_w2'], r1(p['c1_b2']))
    c1 = _node_mid(_scatmax64(m1, pk, cnt),
                   r1(p['bn1_w']), r1(p['bn1_b']))

    gd2, gs2 = _gather_64_64(c1, dst, c1, src)
    m2 = _mix2(gd2, gs2, p['c2_w1'], r1(p['c2_b1']),
               p['c2_w2'], r1(p['c2_b2']))
    hxt = _node_mid2(_scatmax128(m2, pk, cnt),
                     r1(p['bn12_w']), r1(p['bn12_b']),
                     p['c3_linx_w'], r1(p['c3_linx_b']),
                     p['c4_linx_w'], r1(p['c4_linx_b']))

    ghd, ghs = _gather_128_128(hxt, dst, hxt, src)
    a = _attn(ea3, ghd, ghs, p['mlp_edge_w'], r1(p['mlp_edge_b']),
              p['c3_eemb_w'], p['c4_eemb_w'],
              p['c3_att_w'], p['c4_att_w'])
    stats = _stats(a, pk, cnt)
    gst = _gather_32(stats, dst)
    o3, o4 = _wout(ea3, ghs, a, gst, p['mlp_edge_w'], r1(p['mlp_edge_b']),
                   p['c3_eemb_w'], p['c4_eemb_w'],
                   p['c3_lin_w'], p['c4_lin_w'],
                   r1(p['c3_lin_b']), r1(p['c4_lin_b']))
    agg3 = _scatmax192(o3, pk, cnt)
    agg4 = _scatmax192(o4, pk, cnt)

    cls_out, reg_out, cco, cro = _final(
        agg3, agg4, hxt,
        r1(p['bn2_w']), r1(p['bn2_b']),
        r1(p['bn22_w']), r1(p['bn22_b']),
        p['mlp_emb_w'], r1(p['mlp_emb_b']),
        p['mlp_dec_w'], r1(p['mlp_dec_b']))
    return (cls_out, reg_out, cco, cro)


# speedup vs baseline: 1.0029x; 1.0029x over previous
import functools

import jax
import jax.numpy as jnp
from jax import lax
from jax.experimental import pallas as pl
from jax.experimental.pallas import tpu as pltpu
from jax.experimental.pallas import tpu_sc as plsc

N = 10000
E = 160000
NC = 2
NS = 16
NW = NC * NS
NR = 320
NPAD = NW * NR
FLUSH = 8192
PKCAP = 168192
CG = 128
NFULL = 39
NEXTRA = 2
EID_MASK = 0x3FFFF
NRL = N - (NW - 1) * NR
NEGINF = float("-inf")

_mesh = plsc.VectorSubcoreMesh(core_axis_name="c", subcore_axis_name="s",
                               num_cores=NC, num_subcores=NS)
_sc_params = pltpu.CompilerParams(use_tc_tiling_on_sc=False,
                                  needs_layout_passes=False)
_sc_params_tc = pltpu.CompilerParams(use_tc_tiling_on_sc=True,
                                     needs_layout_passes=False)


def _wid():
    return lax.axis_index("s") * NC + lax.axis_index("c")


def _f32(shape):
    return jax.ShapeDtypeStruct(shape, jnp.float32)


def _i32(shape):
    return jax.ShapeDtypeStruct(shape, jnp.int32)


@functools.partial(
    pl.kernel,
    out_type=(_i32((NW, PKCAP)), _i32((NW, 16))),
    mesh=_mesh,
    compiler_params=_sc_params,
    scratch_types=[
        pltpu.VMEM((8000,), jnp.int32),
        pltpu.VMEM((FLUSH + 16,), jnp.int32),
        pltpu.VMEM((16,), jnp.int32),
    ],
)
def _route(dst_hbm, pk_hbm, cnt_hbm, dstbuf, pkbuf, cntbuf):
    wid = _wid()
    lo = wid * NR
    hi = lo + NR
    CH = 8000

    def chunk_body(c, carry):
        ptr0, off0 = carry
        pltpu.sync_copy(dst_hbm.at[pl.ds(pl.multiple_of(c * CH, CH), CH)],
                        dstbuf)

        def vec_body(j, carry2):
            ptr, off = carry2
            v = dstbuf[pl.ds(j * 16, 16)]
            eidv = (c * CH + j * 16) + lax.iota(jnp.int32, 16)
            mask = (v >= lo) & (v < hi)
            packed = eidv | ((v - lo) << 18)
            mv = mask.astype(jnp.int32)
            cum = plsc.cumsum(mv)
            plsc.store_scatter(pkbuf, [ptr + cum - mv], packed, mask=mask)
            ptr = ptr + cum[15]
            flush = ptr >= FLUSH

            @pl.when(flush)
            def _():
                pltpu.sync_copy(
                    pkbuf.at[pl.ds(0, FLUSH)],
                    pk_hbm.at[wid, pl.ds(pl.multiple_of(off, FLUSH), FLUSH)])
                tail = pkbuf[pl.ds(FLUSH, 16)]
                pkbuf[pl.ds(0, 16)] = tail

            ptr = jnp.where(flush, ptr - FLUSH, ptr)
            off = jnp.where(flush, off + FLUSH, off)
            return ptr, off

        return lax.fori_loop(0, CH // 16, vec_body, (ptr0, off0))

    ptr, off = lax.fori_loop(0, E // CH, chunk_body,
                             (jnp.int32(0), jnp.int32(0)))
    pltpu.sync_copy(pkbuf.at[pl.ds(0, FLUSH)],
                    pk_hbm.at[wid, pl.ds(pl.multiple_of(off, FLUSH), FLUSH)])
    cntbuf[...] = jnp.full((16,), off + ptr, jnp.int32)
    pltpu.sync_copy(cntbuf, cnt_hbm.at[wid])


def _make_gather2(d0, d1, tc_tiled=False):
    @functools.partial(
        pl.kernel,
        out_type=(_f32((E, d0)), _f32((E, d1))),
        mesh=_mesh,
        compiler_params=_sc_params_tc if tc_tiled else _sc_params,
        scratch_types=[
            pltpu.VMEM((CG,), jnp.int32),
            pltpu.VMEM((CG,), jnp.int32),
            pltpu.VMEM((CG, d0), jnp.float32),
            pltpu.VMEM((CG, d1), jnp.float32),
            pltpu.SemaphoreType.DMA,
            pltpu.SemaphoreType.DMA,
        ],
    )
    def gather2(t0, i0, t1, i1, o0, o1, idxv0, idxv1, rows0, rows1,
                sem0, sem1):
        wid = _wid()

        def do(b):
            base = pl.multiple_of(b, CG)
            pltpu.sync_copy(i0.at[pl.ds(base, CG)], idxv0)
            pltpu.sync_copy(i1.at[pl.ds(base, CG)], idxv1)
            d0 = pltpu.async_copy(t0.at[idxv0], rows0, sem0)
            d1 = pltpu.async_copy(t1.at[idxv1], rows1, sem1)
            d0.wait()
            pltpu.sync_copy(rows0, o0.at[pl.ds(base, CG)])
            d1.wait()
            pltpu.sync_copy(rows1, o1.at[pl.ds(base, CG)])

        def body(c, _):
            do((wid * NFULL + c) * CG)
            return 0

        lax.fori_loop(0, NFULL, body, 0)

        @pl.when(wid < NEXTRA)
        def _():
            do((NW * NFULL + wid) * CG)

    return gather2


def _make_gather1(d0):
    @functools.partial(
        pl.kernel,
        out_type=_f32((E, d0)),
        mesh=_mesh,
        compiler_params=_sc_params,
        scratch_types=[
            pltpu.VMEM((CG,), jnp.int32),
            pltpu.VMEM((CG, d0), jnp.float32),
            pltpu.SemaphoreType.DMA,
        ],
    )
    def gather1(t0, i0, o0, idxv, rows0, sem):
        wid = _wid()

        def do(b):
            base = pl.multiple_of(b, CG)
            pltpu.sync_copy(i0.at[pl.ds(base, CG)], idxv)
            pltpu.async_copy(t0.at[idxv], rows0, sem).wait()
            pltpu.sync_copy(rows0, o0.at[pl.ds(base, CG)])

        def body(c, _):
            do((wid * NFULL + c) * CG)
            return 0

        lax.fori_loop(0, NFULL, body, 0)

        @pl.when(wid < NEXTRA)
        def _():
            do((NW * NFULL + wid) * CG)

    return gather1


_gather_16_16 = _make_gather2(16, 16)
_gather_64_64 = _make_gather2(64, 64)
_gather_128_128 = _make_gather2(128, 128)
_gather_32 = _make_gather1(32)


def _load_chunk(pk_hbm, wid, c, pkchunk, eidbuf):
    pltpu.sync_copy(pk_hbm.at[wid, pl.ds(pl.multiple_of(c * CG, CG), CG)],
                    pkchunk.at[pl.ds(0, CG)])
    for j in range(CG // 16):
        pv = pkchunk[pl.ds(j * 16, 16)]
        ev = jnp.minimum(pv & EID_MASK, jnp.int32(E - 1))
        eidbuf[pl.ds(j * 16, 16)] = ev


def _make_scatmax(d, tc_tiled=False):
    ngr = d // 16

    @functools.partial(
        pl.kernel,
        out_type=_f32((N, d)),
        mesh=_mesh,
        compiler_params=_sc_params_tc if tc_tiled else _sc_params,
        scratch_types=[
            pltpu.VMEM((NR, d), jnp.float32),
            pltpu.VMEM((CG + 16,), jnp.int32),
            pltpu.VMEM((CG,), jnp.int32),
            pltpu.VMEM((CG, d), jnp.float32),
            pltpu.VMEM((16,), jnp.int32),
            pltpu.SemaphoreType.DMA,
        ],
    )
    def scatmax(m_hbm, pk_hbm, cnt_hbm, out_hbm, acc, pkchunk, eidbuf,
                rows, cntv, sem):
        wid = _wid()

        def init_body(r, _):
            for g in range(ngr):
                acc[r, pl.ds(g * 16, 16)] = jnp.full((16,), NEGINF)
            return 0

        lax.fori_loop(0, NR, init_body, 0)

        pltpu.sync_copy(cnt_hbm.at[wid], cntv)
        cnt = cntv[...][0]
        nch = (cnt + CG - 1) // CG

        def chunk_body(c, _):
            _load_chunk(pk_hbm, wid, c, pkchunk, eidbuf)
            pltpu.async_copy(m_hbm.at[eidbuf], rows, sem).wait()
            jn = jnp.minimum(jnp.int32(CG), cnt - c * CG)

            def edge_body(j, _2):
                dl = pkchunk[pl.ds(j, 16)][0] >> 18
                for g in range(ngr):
                    sl = pl.ds(g * 16, 16)
                    acc[dl, sl] = jnp.maximum(acc[dl, sl], rows[j, sl])
                return 0

            lax.fori_loop(0, jn, edge_body, 0)
            return 0

        lax.fori_loop(0, nch, chunk_body, 0)

        def fix_body(r, _):
            for g in range(ngr):
                sl = pl.ds(g * 16, 16)
                v = acc[r, sl]
                acc[r, sl] = jnp.where(v == NEGINF, jnp.float32(0.0), v)
            return 0

        lax.fori_loop(0, NR, fix_body, 0)
        last = wid == NW - 1

        @pl.when(last)
        def _():
            pltpu.sync_copy(acc.at[pl.ds(0, NRL)],
                            out_hbm.at[pl.ds(wid * NR, NRL)])

        @pl.when(jnp.logical_not(last))
        def _():
            pltpu.sync_copy(acc, out_hbm.at[pl.ds(wid * NR, NR)])

    return scatmax


_scatmax64 = _make_scatmax(64)
_scatmax128 = _make_scatmax(128)
_scatmax192 = _make_scatmax(192)


@functools.partial(
    pl.kernel,
    out_type=_f32((N, 32)),
    mesh=_mesh,
    compiler_params=_sc_params,
    scratch_types=[
        pltpu.VMEM((NR, 32), jnp.float32),
        pltpu.VMEM((CG + 16,), jnp.int32),
        pltpu.VMEM((CG,), jnp.int32),
        pltpu.VMEM((CG, 16), jnp.float32),
        pltpu.VMEM((16,), jnp.int32),
        pltpu.SemaphoreType.DMA,
    ],
)
def _stats(a_hbm, pk_hbm, cnt_hbm, out_hbm, acc, pkchunk, eidbuf, rows,
           cntv, sem):
    wid = _wid()
    lo_m = pl.ds(0, 16)
    lo_s = pl.ds(16, 16)

    def init_body(r, _):
        acc[r, lo_m] = jnp.full((16,), NEGINF)
        acc[r, lo_s] = jnp.zeros((16,), jnp.float32)
        return 0

    lax.fori_loop(0, NR, init_body, 0)

    pltpu.sync_copy(cnt_hbm.at[wid], cntv)
    cnt = cntv[...][0]
    nch = (cnt + CG - 1) // CG

    def max_chunk(c, _):
        _load_chunk(pk_hbm, wid, c, pkchunk, eidbuf)
        pltpu.async_copy(a_hbm.at[eidbuf], rows, sem).wait()
        jn = jnp.minimum(jnp.int32(CG), cnt - c * CG)

        def edge_body(j, _2):
            dl = pkchunk[pl.ds(j, 16)][0] >> 18
            acc[dl, lo_m] = jnp.maximum(acc[dl, lo_m], rows[j, pl.ds(0, 16)])
            return 0

        lax.fori_loop(0, jn, edge_body, 0)
        return 0

    lax.fori_loop(0, nch, max_chunk, 0)

    def sum_chunk(c, _):
        _load_chunk(pk_hbm, wid, c, pkchunk, eidbuf)
        pltpu.async_copy(a_hbm.at[eidbuf], rows, sem).wait()
        jn = jnp.minimum(jnp.int32(CG), cnt - c * CG)

        def edge_body(j, _2):
            dl = pkchunk[pl.ds(j, 16)][0] >> 18
            e = jnp.exp(rows[j, pl.ds(0, 16)] - acc[dl, lo_m])
            acc[dl, lo_s] = acc[dl, lo_s] + e
            return 0

        lax.fori_loop(0, jn, edge_body, 0)
        return 0

    lax.fori_loop(0, nch, sum_chunk, 0)
    last = wid == NW - 1

    @pl.when(last)
    def _():
        pltpu.sync_copy(acc.at[pl.ds(0, NRL)],
                        out_hbm.at[pl.ds(wid * NR, NRL)])

    @pl.when(jnp.logical_not(last))
    def _():
        pltpu.sync_copy(acc, out_hbm.at[pl.ds(wid * NR, NR)])


EBLK = 1000
NEB = E // EBLK


def _eb(d):
    return pl.BlockSpec((EBLK, d), lambda i: (i, 0))


def _full2(shape):
    return pl.BlockSpec(shape, lambda i: (0, 0))


def _dot(a, b):
    return jnp.dot(a, b, preferred_element_type=jnp.float32)


def _bn_relu(v, w, b):
    mu = jnp.mean(v, axis=0, keepdims=True)
    var = jnp.mean((v - mu) * (v - mu), axis=0, keepdims=True)
    return jax.nn.relu((v - mu) / jnp.sqrt(var + 1e-5) * w + b)


def _node_prep_body(x_ref, wn_ref, bn_ref, ne_ref):
    ne_ref[...] = jax.nn.relu(
        _dot(x_ref[...] * 0.01, wn_ref[...]) + bn_ref[...])


def _node_prep(x, wn, bnb):
    return pl.pallas_call(
        _node_prep_body,
        out_shape=_f32((N, 16)),
    )(x, wn, bnb)


def _make_mix(d_in, d_out):
    def body(gd_ref, gs_ref, w1_ref, b1_ref, w2_ref, b2_ref, m_ref):
        xi = gd_ref[...]
        xj = gs_ref[...]
        m_in = jnp.concatenate([xi, xj - xi], axis=1)
        m = jax.nn.relu(_dot(m_in, w1_ref[...]) + b1_ref[...])
        m_ref[...] = _dot(m, w2_ref[...]) + b2_ref[...]

    def mix(gd, gs, w1, b1, w2, b2):
        return pl.pallas_call(
            body,
            grid=(NEB,),
            in_specs=[_eb(d_in), _eb(d_in),
                      _full2((2 * d_in, d_out)), _full2((1, d_out)),
                      _full2((d_out, d_out)), _full2((1, d_out))],
            out_specs=_eb(d_out),
            out_shape=_f32((E, d_out)),
        )(gd, gs, w1, b1, w2, b2)

    return mix


_mix1 = _make_mix(16, 64)
_mix2 = _make_mix(64, 128)


def _node_mid_body(c1_ref, bw_ref, bb_ref, c1o_ref):
    c1o_ref[...] = _bn_relu(c1_ref[...], bw_ref[...], bb_ref[...])


def _node_mid(c1raw, bw, bb):
    return pl.pallas_call(
        _node_mid_body,
        out_shape=_f32((N, 64)),
    )(c1raw, bw, bb)


def _node_mid2_body(c2_ref, bw_ref, bb_ref, lx3_ref, lxb3_ref, lx4_ref,
                    lxb4_ref, hxt_ref):
    h = _bn_relu(c2_ref[...], bw_ref[...], bb_ref[...])
    hx3 = _dot(h, lx3_ref[...]) + lxb3_ref[...]
    hx4 = _dot(h, lx4_ref[...]) + lxb4_ref[...]
    hxt_ref[...] = jnp.concatenate([hx3, hx4], axis=1)


def _node_mid2(c2raw, bw, bb, lx3, lxb3, lx4, lxb4):
    return pl.pallas_call(
        _node_mid2_body,
        out_shape=_f32((N, 128)),
    )(c2raw, bw, bb, lx3, lxb3, lx4, lxb4)


def _edge_e2(ea_ref, we_ref, be_ref, em_ref):
    ea = ea_ref[...]
    ea = jnp.concatenate([ea[:, :1] * 0.01, ea[:, 1:]], axis=1)
    ee = jax.nn.relu(_dot(ea, we_ref[...]) + be_ref[...])
    return jax.nn.leaky_relu(_dot(ee, em_ref[...]), 0.2)


def _attn_body(ea_ref, ghd_ref, ghs_ref, we_ref, be_ref, em3_ref, em4_ref,
               at3_ref, at4_ref, a_ref):
    ghd = ghd_ref[...]
    ghs = ghs_ref[...]
    e23 = _edge_e2(ea_ref, we_ref, be_ref, em3_ref)
    e24 = _edge_e2(ea_ref, we_ref, be_ref, em4_ref)
    c3 = jnp.concatenate([ghd[:, :64], ghs[:, :64], e23], axis=1)
    c4 = jnp.concatenate([ghd[:, 64:], ghs[:, 64:], e24], axis=1)
    a3 = jax.nn.leaky_relu(_dot(c3, at3_ref[...]), 0.2)
    a4 = jax.nn.leaky_relu(_dot(c4, at4_ref[...]), 0.2)
    a_ref[...] = jnp.concatenate(
        [a3, a4, jnp.zeros((EBLK, 10), jnp.float32)], axis=1)


def _attn(ea3, ghd, ghs, we, be, em3, em4, at3, at4):
    return pl.pallas_call(
        _attn_body,
        grid=(NEB,),
        in_specs=[_eb(3), _eb(128), _eb(128),
                  _full2((3, 64)), _full2((1, 64)),
                  _full2((64, 128)), _full2((64, 128)),
                  _full2((256, 3)), _full2((256, 3))],
        out_specs=_eb(16),
        out_shape=_f32((E, 16)),
    )(ea3, ghd, ghs, we, be, em3, em4, at3, at4)


def _wout_body(ea_ref, ghs_ref, a_ref, gst_ref, we_ref, be_ref, em3_ref,
               em4_ref, ln3_ref, ln4_ref, lb3_ref, lb4_ref,
               o3_ref, o4_ref):
    ghs = ghs_ref[...]
    e23 = _edge_e2(ea_ref, we_ref, be_ref, em3_ref)
    e24 = _edge_e2(ea_ref, we_ref, be_ref, em4_ref)
    o3 = _dot(jnp.concatenate([ghs[:, :64], e23], axis=1),
              ln3_ref[...]) + lb3_ref[...]
    o4 = _dot(jnp.concatenate([ghs[:, 64:], e24], axis=1),
              ln4_ref[...]) + lb4_ref[...]
    gst = gst_ref[...]
    w6 = jnp.exp(a_ref[...][:, :6] - gst[:, :6]) / (gst[:, 16:22] + 1e-16)
    o3_ref[...] = jnp.concatenate(
        [o3 * w6[:, 0:1], o3 * w6[:, 1:2], o3 * w6[:, 2:3]], axis=1)
    o4_ref[...] = jnp.concatenate(
        [o4 * w6[:, 3:4], o4 * w6[:, 4:5], o4 * w6[:, 5:6]], axis=1)


def _wout(ea3, ghs, a, gst, we, be, em3, em4, ln3, ln4, lb3, lb4):
    return pl.pallas_call(
        _wout_body,
        grid=(NEB,),
        in_specs=[_eb(3), _eb(128), _eb(16), _eb(32),
                  _full2((3, 64)), _full2((1, 64)),
                  _full2((64, 128)), _full2((64, 128)),
                  _full2((192, 64)), _full2((192, 64)),
                  _full2((1, 64)), _full2((1, 64))],
        out_specs=(_eb(192), _eb(192)),
        out_shape=(_f32((E, 192)), _f32((E, 192))),
    )(ea3, ghs, a, gst, we, be, em3, em4, ln3, ln4, lb3, lb4)


NBLK = 10
NBR = N // NBLK


def _nb(d):
    return pl.BlockSpec((NBR, d), lambda i: (i, 0))


def _bnstats_body(agg3_ref, agg4_ref, hxt_ref, st_ref):
    i = pl.program_id(0)

    @pl.when(i == 0)
    def _():
        st_ref[...] = jnp.zeros_like(st_ref)

    hxt = hxt_ref[...]
    hx3 = hxt[:, :64]
    hx4 = hxt[:, 64:]
    p3 = agg3_ref[...] + jnp.concatenate([hx3, hx3, hx3], axis=1)
    p4 = agg4_ref[...] + jnp.concatenate([hx4, hx4, hx4], axis=1)
    st = jnp.stack([
        jnp.sum(p3, axis=0), jnp.sum(p3 * p3, axis=0),
        jnp.sum(p4, axis=0), jnp.sum(p4 * p4, axis=0)], axis=0)
    st_ref[...] = st_ref[...] + st


def _bnstats(agg3, agg4, hxt):
    return pl.pallas_call(
        _bnstats_body,
        grid=(NBLK,),
        in_specs=[_nb(192), _nb(192), _nb(128)],
        out_specs=pl.BlockSpec((4, 192), lambda i: (0, 0)),
        out_shape=_f32((4, 192)),
    )(agg3, agg4, hxt)


def _final_body(agg3_ref, agg4_ref, hxt_ref, st_ref, b2w_ref,
                b2b_ref, b22w_ref, b22b_ref, ew_ref, eb_ref, dw_ref, db_ref,
                cls_ref, reg_ref, cco_ref, cro_ref):
    st = st_ref[...]
    mu3 = st[0:1] * (1.0 / N)
    var3 = st[1:2] * (1.0 / N) - mu3 * mu3
    mu4 = st[2:3] * (1.0 / N)
    var4 = st[3:4] * (1.0 / N) - mu4 * mu4
    hxt = hxt_ref[...]
    hx3 = hxt[:, :64]
    hx4 = hxt[:, 64:]
    p3 = agg3_ref[...] + jnp.concatenate([hx3, hx3, hx3], axis=1)
    p4 = agg4_ref[...] + jnp.concatenate([hx4, hx4, hx4], axis=1)
    cco = jax.nn.relu(
        (p3 - mu3) / jnp.sqrt(var3 + 1e-5) * b2w_ref[...] + b2b_ref[...])
    cro = jax.nn.relu(
        (p4 - mu4) / jnp.sqrt(var4 + 1e-5) * b22w_ref[...] + b22b_ref[...])
    cco_ref[...] = cco
    cro_ref[...] = cro
    emb3 = jax.nn.relu(_dot(cco, ew_ref[...]) + eb_ref[...])
    emb4 = jax.nn.relu(_dot(cro, ew_ref[...]) + eb_ref[...])
    cls_ref[...] = jax.nn.sigmoid(_dot(emb3, dw_ref[...]) + db_ref[...])
    reg_ref[...] = jax.nn.sigmoid(_dot(emb4, dw_ref[...]) + db_ref[...])


def _final(agg3, agg4, hxt, b2w, b2b, b22w, b22b, ew, ebias, dw, db):
    st = _bnstats(agg3, agg4, hxt)
    return pl.pallas_call(
        _final_body,
        grid=(NBLK,),
        in_specs=[_nb(192), _nb(192), _nb(128),
                  pl.BlockSpec((4, 192), lambda i: (0, 0)),
                  _full2((1, 192)), _full2((1, 192)),
                  _full2((1, 192)), _full2((1, 192)),
                  _full2((192, 64)), _full2((1, 64)),
                  _full2((64, 4)), _full2((1, 4))],
        out_specs=(_nb(4), _nb(4), _nb(192), _nb(192)),
        out_shape=(_f32((N, 4)), _f32((N, 4)),
                   _f32((N, 192)), _f32((N, 192))),
    )(agg3, agg4, hxt, st, b2w, b2b, b22w, b22b, ew, ebias, dw, db)


def kernel(x, edge_index, edge_attr, params):
    p = params
    src = edge_index[0]
    dst = edge_index[1]
    ea3 = edge_attr[:, 5:8]

    def r1(v):
        return v.reshape(1, -1)

    pk, cnt = _route(dst)

    ne = _node_prep(x, p['mlp_node_w'], r1(p['mlp_node_b']))
    gd1, gs1 = _gather_16_16(ne, dst, ne, src)
    m1 = _mix1(gd1, gs1, p['c1_w1'], r1(p['c1_b1']),
               p['c1_w2'], r1(p['c1_b2']))
    c1 = _node_mid(_scatmax64(m1, pk, cnt),
                   r1(p['bn1_w']), r1(p['bn1_b']))

    gd2, gs2 = _gather_64_64(c1, dst, c1, src)
    m2 = _mix2(gd2, gs2, p['c2_w1'], r1(p['c2_b1']),
               p['c2_w2'], r1(p['c2_b2']))
    hxt = _node_mid2(_scatmax128(m2, pk, cnt),
                     r1(p['bn12_w']), r1(p['bn12_b']),
                     p['c3_linx_w'], r1(p['c3_linx_b']),
                     p['c4_linx_w'], r1(p['c4_linx_b']))

    ghd, ghs = _gather_128_128(hxt, dst, hxt, src)
    a = _attn(ea3, ghd, ghs, p['mlp_edge_w'], r1(p['mlp_edge_b']),
              p['c3_eemb_w'], p['c4_eemb_w'],
              p['c3_att_w'], p['c4_att_w'])
    stats = _stats(a, pk, cnt)
    gst = _gather_32(stats, dst)
    o3, o4 = _wout(ea3, ghs, a, gst, p['mlp_edge_w'], r1(p['mlp_edge_b']),
                   p['c3_eemb_w'], p['c4_eemb_w'],
                   p['c3_lin_w'], p['c4_lin_w'],
                   r1(p['c3_lin_b']), r1(p['c4_lin_b']))
    agg3 = _scatmax192(o3, pk, cnt)
    agg4 = _scatmax192(o4, pk, cnt)

    cls_out, reg_out, cco, cro = _final(
        agg3, agg4, hxt,
        r1(p['bn2_w']), r1(p['bn2_b']),
        r1(p['bn22_w']), r1(p['bn22_b']),
        p['mlp_emb_w'], r1(p['mlp_emb_b']),
        p['mlp_dec_w'], r1(p['mlp_dec_b']))
    return (cls_out, reg_out, cco, cro)


# speedup vs baseline: 1.0119x; 1.0090x over previous
import functools

import jax
import jax.numpy as jnp
from jax import lax
from jax.experimental import pallas as pl
from jax.experimental.pallas import tpu as pltpu
from jax.experimental.pallas import tpu_sc as plsc

N = 10000
E = 160000
NC = 2
NS = 16
NW = NC * NS
NR = 313
NPAD = NW * NR
FLUSH = 8192
PKCAP = 168192
CG = 128
NFULL = 39
NEXTRA = 2
EID_MASK = 0x3FFFF
NRL = N - (NW - 1) * NR
NEGINF = float("-inf")

_mesh = plsc.VectorSubcoreMesh(core_axis_name="c", subcore_axis_name="s",
                               num_cores=NC, num_subcores=NS)
_sc_params = pltpu.CompilerParams(use_tc_tiling_on_sc=False,
                                  needs_layout_passes=False)
_sc_params_tc = pltpu.CompilerParams(use_tc_tiling_on_sc=True,
                                     needs_layout_passes=False)


def _wid():
    return lax.axis_index("s") * NC + lax.axis_index("c")


def _f32(shape):
    return jax.ShapeDtypeStruct(shape, jnp.float32)


def _i32(shape):
    return jax.ShapeDtypeStruct(shape, jnp.int32)


@functools.partial(
    pl.kernel,
    out_type=(_i32((NW, PKCAP)), _i32((NW, 16))),
    mesh=_mesh,
    compiler_params=_sc_params,
    scratch_types=[
        pltpu.VMEM((8000,), jnp.int32),
        pltpu.VMEM((FLUSH + 16,), jnp.int32),
        pltpu.VMEM((16,), jnp.int32),
    ],
)
def _route(dst_hbm, pk_hbm, cnt_hbm, dstbuf, pkbuf, cntbuf):
    wid = _wid()
    lo = wid * NR
    hi = lo + NR
    CH = 8000

    def chunk_body(c, carry):
        ptr0, off0 = carry
        pltpu.sync_copy(dst_hbm.at[pl.ds(pl.multiple_of(c * CH, CH), CH)],
                        dstbuf)

        def vec_body(j, carry2):
            ptr, off = carry2
            v = dstbuf[pl.ds(j * 16, 16)]
            eidv = (c * CH + j * 16) + lax.iota(jnp.int32, 16)
            mask = (v >= lo) & (v < hi)
            packed = eidv | ((v - lo) << 18)
            mv = mask.astype(jnp.int32)
            cum = plsc.cumsum(mv)
            plsc.store_scatter(pkbuf, [ptr + cum - mv], packed, mask=mask)
            ptr = ptr + cum[15]
            flush = ptr >= FLUSH

            @pl.when(flush)
            def _():
                pltpu.sync_copy(
                    pkbuf.at[pl.ds(0, FLUSH)],
                    pk_hbm.at[wid, pl.ds(pl.multiple_of(off, FLUSH), FLUSH)])
                tail = pkbuf[pl.ds(FLUSH, 16)]
                pkbuf[pl.ds(0, 16)] = tail

            ptr = jnp.where(flush, ptr - FLUSH, ptr)
            off = jnp.where(flush, off + FLUSH, off)
            return ptr, off

        return lax.fori_loop(0, CH // 16, vec_body, (ptr0, off0))

    ptr, off = lax.fori_loop(0, E // CH, chunk_body,
                             (jnp.int32(0), jnp.int32(0)))
    pltpu.sync_copy(pkbuf.at[pl.ds(0, FLUSH)],
                    pk_hbm.at[wid, pl.ds(pl.multiple_of(off, FLUSH), FLUSH)])
    cntbuf[...] = jnp.full((16,), off + ptr, jnp.int32)
    pltpu.sync_copy(cntbuf, cnt_hbm.at[wid])


def _make_gather2(d0, d1, tc_tiled=False):
    @functools.partial(
        pl.kernel,
        out_type=(_f32((E, d0)), _f32((E, d1))),
        mesh=_mesh,
        compiler_params=_sc_params_tc if tc_tiled else _sc_params,
        scratch_types=[
            pltpu.VMEM((CG,), jnp.int32),
            pltpu.VMEM((CG,), jnp.int32),
            pltpu.VMEM((CG, d0), jnp.float32),
            pltpu.VMEM((CG, d1), jnp.float32),
            pltpu.SemaphoreType.DMA,
            pltpu.SemaphoreType.DMA,
        ],
    )
    def gather2(t0, i0, t1, i1, o0, o1, idxv0, idxv1, rows0, rows1,
                sem0, sem1):
        wid = _wid()

        def do(b):
            base = pl.multiple_of(b, CG)
            pltpu.sync_copy(i0.at[pl.ds(base, CG)], idxv0)
            pltpu.sync_copy(i1.at[pl.ds(base, CG)], idxv1)
            d0 = pltpu.async_copy(t0.at[idxv0], rows0, sem0)
            d1 = pltpu.async_copy(t1.at[idxv1], rows1, sem1)
            d0.wait()
            pltpu.sync_copy(rows0, o0.at[pl.ds(base, CG)])
            d1.wait()
            pltpu.sync_copy(rows1, o1.at[pl.ds(base, CG)])

        def body(c, _):
            do((wid * NFULL + c) * CG)
            return 0

        lax.fori_loop(0, NFULL, body, 0)

        @pl.when(wid < NEXTRA)
        def _():
            do((NW * NFULL + wid) * CG)

    return gather2


def _make_gather1(d0):
    @functools.partial(
        pl.kernel,
        out_type=_f32((E, d0)),
        mesh=_mesh,
        compiler_params=_sc_params,
        scratch_types=[
            pltpu.VMEM((CG,), jnp.int32),
            pltpu.VMEM((CG, d0), jnp.float32),
            pltpu.SemaphoreType.DMA,
        ],
    )
    def gather1(t0, i0, o0, idxv, rows0, sem):
        wid = _wid()

        def do(b):
            base = pl.multiple_of(b, CG)
            pltpu.sync_copy(i0.at[pl.ds(base, CG)], idxv)
            pltpu.async_copy(t0.at[idxv], rows0, sem).wait()
            pltpu.sync_copy(rows0, o0.at[pl.ds(base, CG)])

        def body(c, _):
            do((wid * NFULL + c) * CG)
            return 0

        lax.fori_loop(0, NFULL, body, 0)

        @pl.when(wid < NEXTRA)
        def _():
            do((NW * NFULL + wid) * CG)

    return gather1


_gather_16_16 = _make_gather2(16, 16)
_gather_64_64 = _make_gather2(64, 64)
_gather_128_128 = _make_gather2(128, 128)
_gather_32 = _make_gather1(32)


def _load_chunk(pk_hbm, wid, c, pkchunk, eidbuf):
    pltpu.sync_copy(pk_hbm.at[wid, pl.ds(pl.multiple_of(c * CG, CG), CG)],
                    pkchunk.at[pl.ds(0, CG)])
    for j in range(CG // 16):
        pv = pkchunk[pl.ds(j * 16, 16)]
        ev = jnp.minimum(pv & EID_MASK, jnp.int32(E - 1))
        eidbuf[pl.ds(j * 16, 16)] = ev


def _make_scatmax(d, tc_tiled=False):
    ngr = d // 16

    @functools.partial(
        pl.kernel,
        out_type=_f32((N, d)),
        mesh=_mesh,
        compiler_params=_sc_params_tc if tc_tiled else _sc_params,
        scratch_types=[
            pltpu.VMEM((NR, d), jnp.float32),
            pltpu.VMEM((CG + 16,), jnp.int32),
            pltpu.VMEM((CG,), jnp.int32),
            pltpu.VMEM((CG, d), jnp.float32),
            pltpu.VMEM((16,), jnp.int32),
            pltpu.SemaphoreType.DMA,
        ],
    )
    def scatmax(m_hbm, pk_hbm, cnt_hbm, out_hbm, acc, pkchunk, eidbuf,
                rows, cntv, sem):
        wid = _wid()

        def init_body(r, _):
            for g in range(ngr):
                acc[r, pl.ds(g * 16, 16)] = jnp.full((16,), NEGINF)
            return 0

        lax.fori_loop(0, NR, init_body, 0)

        pltpu.sync_copy(cnt_hbm.at[wid], cntv)
        cnt = cntv[...][0]
        nch = (cnt + CG - 1) // CG

        def chunk_body(c, _):
            _load_chunk(pk_hbm, wid, c, pkchunk, eidbuf)
            pltpu.async_copy(m_hbm.at[eidbuf], rows, sem).wait()
            jn = jnp.minimum(jnp.int32(CG), cnt - c * CG)

            def edge_body(j, _2):
                dl = pkchunk[pl.ds(j, 16)][0] >> 18
                for g in range(ngr):
                    sl = pl.ds(g * 16, 16)
                    acc[dl, sl] = jnp.maximum(acc[dl, sl], rows[j, sl])
                return 0

            lax.fori_loop(0, jn, edge_body, 0)
            return 0

        lax.fori_loop(0, nch, chunk_body, 0)

        def fix_body(r, _):
            for g in range(ngr):
                sl = pl.ds(g * 16, 16)
                v = acc[r, sl]
                acc[r, sl] = jnp.where(v == NEGINF, jnp.float32(0.0), v)
            return 0

        lax.fori_loop(0, NR, fix_body, 0)
        last = wid == NW - 1

        @pl.when(last)
        def _():
            pltpu.sync_copy(acc.at[pl.ds(0, NRL)],
                            out_hbm.at[pl.ds(wid * NR, NRL)])

        @pl.when(jnp.logical_not(last))
        def _():
            pltpu.sync_copy(acc, out_hbm.at[pl.ds(wid * NR, NR)])

    return scatmax


_scatmax64 = _make_scatmax(64)
_scatmax128 = _make_scatmax(128)
_scatmax192 = _make_scatmax(192)


@functools.partial(
    pl.kernel,
    out_type=_f32((N, 32)),
    mesh=_mesh,
    compiler_params=_sc_params,
    scratch_types=[
        pltpu.VMEM((NR, 32), jnp.float32),
        pltpu.VMEM((CG + 16,), jnp.int32),
        pltpu.VMEM((CG,), jnp.int32),
        pltpu.VMEM((CG, 16), jnp.float32),
        pltpu.VMEM((16,), jnp.int32),
        pltpu.SemaphoreType.DMA,
    ],
)
def _stats(a_hbm, pk_hbm, cnt_hbm, out_hbm, acc, pkchunk, eidbuf, rows,
           cntv, sem):
    wid = _wid()
    lo_m = pl.ds(0, 16)
    lo_s = pl.ds(16, 16)

    def init_body(r, _):
        acc[r, lo_m] = jnp.full((16,), NEGINF)
        acc[r, lo_s] = jnp.zeros((16,), jnp.float32)
        return 0

    lax.fori_loop(0, NR, init_body, 0)

    pltpu.sync_copy(cnt_hbm.at[wid], cntv)
    cnt = cntv[...][0]
    nch = (cnt + CG - 1) // CG

    def max_chunk(c, _):
        _load_chunk(pk_hbm, wid, c, pkchunk, eidbuf)
        pltpu.async_copy(a_hbm.at[eidbuf], rows, sem).wait()
        jn = jnp.minimum(jnp.int32(CG), cnt - c * CG)

        def edge_body(j, _2):
            dl = pkchunk[pl.ds(j, 16)][0] >> 18
            acc[dl, lo_m] = jnp.maximum(acc[dl, lo_m], rows[j, pl.ds(0, 16)])
            return 0

        lax.fori_loop(0, jn, edge_body, 0)
        return 0

    lax.fori_loop(0, nch, max_chunk, 0)

    def sum_chunk(c, _):
        _load_chunk(pk_hbm, wid, c, pkchunk, eidbuf)
        pltpu.async_copy(a_hbm.at[eidbuf], rows, sem).wait()
        jn = jnp.minimum(jnp.int32(CG), cnt - c * CG)

        def edge_body(j, _2):
            dl = pkchunk[pl.ds(j, 16)][0] >> 18
            e = jnp.exp(rows[j, pl.ds(0, 16)] - acc[dl, lo_m])
            acc[dl, lo_s] = acc[dl, lo_s] + e
            return 0

        lax.fori_loop(0, jn, edge_body, 0)
        return 0

    lax.fori_loop(0, nch, sum_chunk, 0)
    last = wid == NW - 1

    @pl.when(last)
    def _():
        pltpu.sync_copy(acc.at[pl.ds(0, NRL)],
                        out_hbm.at[pl.ds(wid * NR, NRL)])

    @pl.when(jnp.logical_not(last))
    def _():
        pltpu.sync_copy(acc, out_hbm.at[pl.ds(wid * NR, NR)])


EBLK = 1000
NEB = E // EBLK


def _eb(d):
    return pl.BlockSpec((EBLK, d), lambda i: (i, 0))


def _full2(shape):
    return pl.BlockSpec(shape, lambda i: (0, 0))


def _dot(a, b):
    return jnp.dot(a, b, preferred_element_type=jnp.float32)


def _bn_relu(v, w, b):
    mu = jnp.mean(v, axis=0, keepdims=True)
    var = jnp.mean((v - mu) * (v - mu), axis=0, keepdims=True)
    return jax.nn.relu((v - mu) / jnp.sqrt(var + 1e-5) * w + b)


def _node_prep_body(x_ref, wn_ref, bn_ref, ne_ref):
    ne_ref[...] = jax.nn.relu(
        _dot(x_ref[...] * 0.01, wn_ref[...]) + bn_ref[...])


def _node_prep(x, wn, bnb):
    return pl.pallas_call(
        _node_prep_body,
        out_shape=_f32((N, 16)),
    )(x, wn, bnb)


def _make_mix(d_in, d_out):
    def body(gd_ref, gs_ref, w1_ref, b1_ref, w2_ref, b2_ref, m_ref):
        xi = gd_ref[...]
        xj = gs_ref[...]
        m_in = jnp.concatenate([xi, xj - xi], axis=1)
        m = jax.nn.relu(_dot(m_in, w1_ref[...]) + b1_ref[...])
        m_ref[...] = _dot(m, w2_ref[...]) + b2_ref[...]

    def mix(gd, gs, w1, b1, w2, b2):
        return pl.pallas_call(
            body,
            grid=(NEB,),
            in_specs=[_eb(d_in), _eb(d_in),
                      _full2((2 * d_in, d_out)), _full2((1, d_out)),
                      _full2((d_out, d_out)), _full2((1, d_out))],
            out_specs=_eb(d_out),
            out_shape=_f32((E, d_out)),
        )(gd, gs, w1, b1, w2, b2)

    return mix


_mix1 = _make_mix(16, 64)
_mix2 = _make_mix(64, 128)


def _node_mid_body(c1_ref, bw_ref, bb_ref, c1o_ref):
    c1o_ref[...] = _bn_relu(c1_ref[...], bw_ref[...], bb_ref[...])


def _node_mid(c1raw, bw, bb):
    return pl.pallas_call(
        _node_mid_body,
        out_shape=_f32((N, 64)),
    )(c1raw, bw, bb)


def _node_mid2_body(c2_ref, bw_ref, bb_ref, lx3_ref, lxb3_ref, lx4_ref,
                    lxb4_ref, hxt_ref):
    h = _bn_relu(c2_ref[...], bw_ref[...], bb_ref[...])
    hx3 = _dot(h, lx3_ref[...]) + lxb3_ref[...]
    hx4 = _dot(h, lx4_ref[...]) + lxb4_ref[...]
    hxt_ref[...] = jnp.concatenate([hx3, hx4], axis=1)


def _node_mid2(c2raw, bw, bb, lx3, lxb3, lx4, lxb4):
    return pl.pallas_call(
        _node_mid2_body,
        out_shape=_f32((N, 128)),
    )(c2raw, bw, bb, lx3, lxb3, lx4, lxb4)


def _edge_e2(ea_ref, we_ref, be_ref, em_ref):
    ea = ea_ref[...]
    ea = jnp.concatenate([ea[:, :1] * 0.01, ea[:, 1:]], axis=1)
    ee = jax.nn.relu(_dot(ea, we_ref[...]) + be_ref[...])
    return jax.nn.leaky_relu(_dot(ee, em_ref[...]), 0.2)


def _attn_body(ea_ref, ghd_ref, ghs_ref, we_ref, be_ref, em3_ref, em4_ref,
               at3_ref, at4_ref, a_ref):
    ghd = ghd_ref[...]
    ghs = ghs_ref[...]
    e23 = _edge_e2(ea_ref, we_ref, be_ref, em3_ref)
    e24 = _edge_e2(ea_ref, we_ref, be_ref, em4_ref)
    c3 = jnp.concatenate([ghd[:, :64], ghs[:, :64], e23], axis=1)
    c4 = jnp.concatenate([ghd[:, 64:], ghs[:, 64:], e24], axis=1)
    a3 = jax.nn.leaky_relu(_dot(c3, at3_ref[...]), 0.2)
    a4 = jax.nn.leaky_relu(_dot(c4, at4_ref[...]), 0.2)
    a_ref[...] = jnp.concatenate(
        [a3, a4, jnp.zeros((EBLK, 10), jnp.float32)], axis=1)


def _attn(ea3, ghd, ghs, we, be, em3, em4, at3, at4):
    return pl.pallas_call(
        _attn_body,
        grid=(NEB,),
        in_specs=[_eb(3), _eb(128), _eb(128),
                  _full2((3, 64)), _full2((1, 64)),
                  _full2((64, 128)), _full2((64, 128)),
                  _full2((256, 3)), _full2((256, 3))],
        out_specs=_eb(16),
        out_shape=_f32((E, 16)),
    )(ea3, ghd, ghs, we, be, em3, em4, at3, at4)


def _wout_body(ea_ref, ghs_ref, a_ref, gst_ref, we_ref, be_ref, em3_ref,
               em4_ref, ln3_ref, ln4_ref, lb3_ref, lb4_ref,
               o3_ref, o4_ref):
    ghs = ghs_ref[...]
    e23 = _edge_e2(ea_ref, we_ref, be_ref, em3_ref)
    e24 = _edge_e2(ea_ref, we_ref, be_ref, em4_ref)
    o3 = _dot(jnp.concatenate([ghs[:, :64], e23], axis=1),
              ln3_ref[...]) + lb3_ref[...]
    o4 = _dot(jnp.concatenate([ghs[:, 64:], e24], axis=1),
              ln4_ref[...]) + lb4_ref[...]
    gst = gst_ref[...]
    w6 = jnp.exp(a_ref[...][:, :6] - gst[:, :6]) / (gst[:, 16:22] + 1e-16)
    o3_ref[...] = jnp.concatenate(
        [o3 * w6[:, 0:1], o3 * w6[:, 1:2], o3 * w6[:, 2:3]], axis=1)
    o4_ref[...] = jnp.concatenate(
        [o4 * w6[:, 3:4], o4 * w6[:, 4:5], o4 * w6[:, 5:6]], axis=1)


def _wout(ea3, ghs, a, gst, we, be, em3, em4, ln3, ln4, lb3, lb4):
    return pl.pallas_call(
        _wout_body,
        grid=(NEB,),
        in_specs=[_eb(3), _eb(128), _eb(16), _eb(32),
                  _full2((3, 64)), _full2((1, 64)),
                  _full2((64, 128)), _full2((64, 128)),
                  _full2((192, 64)), _full2((192, 64)),
                  _full2((1, 64)), _full2((1, 64))],
        out_specs=(_eb(192), _eb(192)),
        out_shape=(_f32((E, 192)), _f32((E, 192))),
    )(ea3, ghs, a, gst, we, be, em3, em4, ln3, ln4, lb3, lb4)


NBLK = 10
NBR = N // NBLK


def _nb(d):
    return pl.BlockSpec((NBR, d), lambda i: (i, 0))


def _bnstats_body(agg3_ref, agg4_ref, hxt_ref, st_ref):
    i = pl.program_id(0)

    @pl.when(i == 0)
    def _():
        st_ref[...] = jnp.zeros_like(st_ref)

    hxt = hxt_ref[...]
    hx3 = hxt[:, :64]
    hx4 = hxt[:, 64:]
    p3 = agg3_ref[...] + jnp.concatenate([hx3, hx3, hx3], axis=1)
    p4 = agg4_ref[...] + jnp.concatenate([hx4, hx4, hx4], axis=1)
    st = jnp.stack([
        jnp.sum(p3, axis=0), jnp.sum(p3 * p3, axis=0),
        jnp.sum(p4, axis=0), jnp.sum(p4 * p4, axis=0)], axis=0)
    st_ref[...] = st_ref[...] + st


def _bnstats(agg3, agg4, hxt):
    return pl.pallas_call(
        _bnstats_body,
        grid=(NBLK,),
        in_specs=[_nb(192), _nb(192), _nb(128)],
        out_specs=pl.BlockSpec((4, 192), lambda i: (0, 0)),
        out_shape=_f32((4, 192)),
    )(agg3, agg4, hxt)


def _final_body(agg3_ref, agg4_ref, hxt_ref, st_ref, b2w_ref,
                b2b_ref, b22w_ref, b22b_ref, ew_ref, eb_ref, dw_ref, db_ref,
                cls_ref, reg_ref, cco_ref, cro_ref):
    st = st_ref[...]
    mu3 = st[0:1] * (1.0 / N)
    var3 = st[1:2] * (1.0 / N) - mu3 * mu3
    mu4 = st[2:3] * (1.0 / N)
    var4 = st[3:4] * (1.0 / N) - mu4 * mu4
    hxt = hxt_ref[...]
    hx3 = hxt[:, :64]
    hx4 = hxt[:, 64:]
    p3 = agg3_ref[...] + jnp.concatenate([hx3, hx3, hx3], axis=1)
    p4 = agg4_ref[...] + jnp.concatenate([hx4, hx4, hx4], axis=1)
    cco = jax.nn.relu(
        (p3 - mu3) / jnp.sqrt(var3 + 1e-5) * b2w_ref[...] + b2b_ref[...])
    cro = jax.nn.relu(
        (p4 - mu4) / jnp.sqrt(var4 + 1e-5) * b22w_ref[...] + b22b_ref[...])
    cco_ref[...] = cco
    cro_ref[...] = cro
    emb3 = jax.nn.relu(_dot(cco, ew_ref[...]) + eb_ref[...])
    emb4 = jax.nn.relu(_dot(cro, ew_ref[...]) + eb_ref[...])
    cls_ref[...] = jax.nn.sigmoid(_dot(emb3, dw_ref[...]) + db_ref[...])
    reg_ref[...] = jax.nn.sigmoid(_dot(emb4, dw_ref[...]) + db_ref[...])


def _final(agg3, agg4, hxt, b2w, b2b, b22w, b22b, ew, ebias, dw, db):
    st = _bnstats(agg3, agg4, hxt)
    return pl.pallas_call(
        _final_body,
        grid=(NBLK,),
        in_specs=[_nb(192), _nb(192), _nb(128),
                  pl.BlockSpec((4, 192), lambda i: (0, 0)),
                  _full2((1, 192)), _full2((1, 192)),
                  _full2((1, 192)), _full2((1, 192)),
                  _full2((192, 64)), _full2((1, 64)),
                  _full2((64, 4)), _full2((1, 4))],
        out_specs=(_nb(4), _nb(4), _nb(192), _nb(192)),
        out_shape=(_f32((N, 4)), _f32((N, 4)),
                   _f32((N, 192)), _f32((N, 192))),
    )(agg3, agg4, hxt, st, b2w, b2b, b22w, b22b, ew, ebias, dw, db)


def kernel(x, edge_index, edge_attr, params):
    p = params
    src = edge_index[0]
    dst = edge_index[1]
    ea3 = edge_attr[:, 5:8]

    def r1(v):
        return v.reshape(1, -1)

    pk, cnt = _route(dst)

    ne = _node_prep(x, p['mlp_node_w'], r1(p['mlp_node_b']))
    gd1, gs1 = _gather_16_16(ne, dst, ne, src)
    m1 = _mix1(gd1, gs1, p['c1_w1'], r1(p['c1_b1']),
               p['c1_w2'], r1(p['c1_b2']))
    c1 = _node_mid(_scatmax64(m1, pk, cnt),
                   r1(p['bn1_w']), r1(p['bn1_b']))

    gd2, gs2 = _gather_64_64(c1, dst, c1, src)
    m2 = _mix2(gd2, gs2, p['c2_w1'], r1(p['c2_b1']),
               p['c2_w2'], r1(p['c2_b2']))
    hxt = _node_mid2(_scatmax128(m2, pk, cnt),
                     r1(p['bn12_w']), r1(p['bn12_b']),
                     p['c3_linx_w'], r1(p['c3_linx_b']),
                     p['c4_linx_w'], r1(p['c4_linx_b']))

    ghd, ghs = _gather_128_128(hxt, dst, hxt, src)
    a = _attn(ea3, ghd, ghs, p['mlp_edge_w'], r1(p['mlp_edge_b']),
              p['c3_eemb_w'], p['c4_eemb_w'],
              p['c3_att_w'], p['c4_att_w'])
    stats = _stats(a, pk, cnt)
    gst = _gather_32(stats, dst)
    o3, o4 = _wout(ea3, ghs, a, gst, p['mlp_edge_w'], r1(p['mlp_edge_b']),
                   p['c3_eemb_w'], p['c4_eemb_w'],
                   p['c3_lin_w'], p['c4_lin_w'],
                   r1(p['c3_lin_b']), r1(p['c4_lin_b']))
    agg3 = _scatmax192(o3, pk, cnt)
    agg4 = _scatmax192(o4, pk, cnt)

    cls_out, reg_out, cco, cro = _final(
        agg3, agg4, hxt,
        r1(p['bn2_w']), r1(p['bn2_b']),
        r1(p['bn22_w']), r1(p['bn22_b']),
        p['mlp_emb_w'], r1(p['mlp_emb_b']),
        p['mlp_dec_w'], r1(p['mlp_dec_b']))
    return (cls_out, reg_out, cco, cro)
